# baseline (speedup 1.0000x reference)
import functools
import math

import jax
import jax.numpy as jnp
from jax import lax
from jax.experimental import pallas as pl
from jax.experimental.pallas import tpu as pltpu
from jax.experimental.pallas import tpu_sc as plsc

NUM_LAYERS = 4
ALPHA = 0.1
THETA = 0.5
NUM_GRAPHS = 64

_NC = 2
_NS = 16
_NW = _NC * _NS
_CH = 64
_BT = 256


def _sc_mesh():
    return plsc.VectorSubcoreMesh(core_axis_name="c", subcore_axis_name="s")


def _deg_counts_kernel(npad, kd, kb, gpad):
    @functools.partial(
        pl.kernel,
        out_type=(jax.ShapeDtypeStruct((_NC, npad), jnp.float32),
                  jax.ShapeDtypeStruct((_NC, gpad), jnp.float32)),
        mesh=_sc_mesh(),
        scratch_types=[
            pltpu.VMEM((kd, _CH), jnp.int32),
            pltpu.VMEM((kb, _CH), jnp.int32),
            pltpu.VMEM((_CH,), jnp.float32),
            pltpu.VMEM_SHARED((npad,), jnp.float32),
            pltpu.VMEM_SHARED((gpad,), jnp.float32),
        ],
    )
    def k(col_hbm, bat_hbm, zeros_hbm, deg_out, cnt_out,
          cidx, bidx, ones_v, deg_sh, cnt_sh):
        cid = lax.axis_index("c")
        sid = lax.axis_index("s")
        wid = cid * _NS + sid
        rp = npad // _NS
        pltpu.sync_copy(zeros_hbm.at[pl.ds(sid * rp, rp)],
                        deg_sh.at[pl.ds(sid * rp, rp)])

        @pl.when(sid == 0)
        def _():
            pltpu.sync_copy(zeros_hbm.at[pl.ds(0, gpad)], cnt_sh)
        pltpu.sync_copy(col_hbm.at[wid], cidx)
        pltpu.sync_copy(bat_hbm.at[wid], bidx)
        for j in range(_CH // 16):
            ones_v[pl.ds(j * 16, 16)] = jnp.ones((16,), jnp.float32)
        plsc.subcore_barrier()

        def dbody(j, c):
            pltpu.sync_copy(ones_v, deg_sh.at[cidx.at[j]], add=True)
            return c
        lax.fori_loop(0, kd, dbody, 0)

        def bbody(j, c):
            pltpu.sync_copy(ones_v, cnt_sh.at[bidx.at[j]], add=True)
            return c
        lax.fori_loop(0, kb, bbody, 0)

        plsc.subcore_barrier()
        pltpu.sync_copy(deg_sh.at[pl.ds(sid * rp, rp)],
                        deg_out.at[cid, pl.ds(sid * rp, rp)])

        @pl.when(sid == 0)
        def _():
            pltpu.sync_copy(cnt_sh, cnt_out.at[cid])

    return k


def _scatter_rows_kernel(msh, mpad, k):
    assert k % 8 == 0 and msh % _NS == 0 and msh <= mpad

    @functools.partial(
        pl.kernel,
        out_type=jax.ShapeDtypeStruct((_NC, mpad, 128), jnp.float32),
        mesh=_sc_mesh(),
        scratch_types=[
            pltpu.VMEM((2, 4, 2, _CH), jnp.int32),
            pltpu.VMEM((2, _CH, 128), jnp.float32),
            pltpu.VMEM_SHARED((msh, 128), jnp.float32),
            pltpu.SemaphoreType.DMA,
            pltpu.SemaphoreType.DMA,
            pltpu.SemaphoreType.DMA,
            pltpu.SemaphoreType.DMA,
        ],
    )
    def kfn(tab_hbm, idx_hbm, zeros_hbm, out_hbm,
            idxr, rows2, agg_sh, semr0, semr1, semi0, semi1):
        cid = lax.axis_index("c")
        sid = lax.axis_index("s")
        wid = cid * _NS + sid
        rp = msh // _NS
        pltpu.sync_copy(zeros_hbm.at[pl.ds(sid * rp, rp)],
                        agg_sh.at[pl.ds(sid * rp, rp)])
        plsc.subcore_barrier()

        semr = (semr0, semr1)

        def wait_idx(slot, sem):
            pltpu.make_async_copy(idx_hbm.at[wid, pl.ds(0, 4)],
                                  idxr.at[slot], sem).wait()

        def chunk(slot, j, rs, gslot, gj):
            pltpu.make_async_copy(tab_hbm.at[idxr.at[slot, j, 0]],
                                  rows2.at[rs], semr[rs]).wait()
            pltpu.sync_copy(rows2.at[rs], agg_sh.at[idxr.at[slot, j, 1]],
                            add=True)
            pltpu.async_copy(tab_hbm.at[idxr.at[gslot, gj, 0]],
                             rows2.at[rs], semr[rs])

        pltpu.sync_copy(idx_hbm.at[wid, pl.ds(0, 4)], idxr.at[0])
        pltpu.async_copy(idx_hbm.at[wid, pl.ds(4, 4)], idxr.at[1], semi1)
        pltpu.async_copy(tab_hbm.at[idxr.at[0, 0, 0]], rows2.at[0], semr0)
        pltpu.async_copy(tab_hbm.at[idxr.at[0, 1, 0]], rows2.at[1], semr1)

        def body(bb, c):
            b8 = bb * 8
            chunk(0, 0, 0, 0, 2)
            chunk(0, 1, 1, 0, 3)
            wait_idx(1, semi1)
            chunk(0, 2, 0, 1, 0)
            chunk(0, 3, 1, 1, 1)
            pltpu.async_copy(idx_hbm.at[wid, pl.ds(b8 + 8, 4)],
                             idxr.at[0], semi0)
            chunk(1, 0, 0, 1, 2)
            chunk(1, 1, 1, 1, 3)
            wait_idx(0, semi0)
            chunk(1, 2, 0, 0, 0)
            chunk(1, 3, 1, 0, 1)
            pltpu.async_copy(idx_hbm.at[wid, pl.ds(b8 + 12, 4)],
                             idxr.at[1], semi1)
            return c
        lax.fori_loop(0, k // 8, body, 0)

        pltpu.make_async_copy(tab_hbm.at[idxr.at[0, 0, 0]],
                              rows2.at[0], semr0).wait()
        pltpu.make_async_copy(tab_hbm.at[idxr.at[0, 1, 0]],
                              rows2.at[1], semr1).wait()
        wait_idx(1, semi1)

        plsc.subcore_barrier()
        pltpu.sync_copy(agg_sh.at[pl.ds(sid * rp, rp)],
                        out_hbm.at[cid, pl.ds(sid * rp, rp)])

    return kfn


def _lin0_call(xp, w, b, degp, npad):
    nblk = npad // _BT

    def body(x_ref, w_ref, b_ref, deg_ref, h_ref, hs_ref, db_ref):
        d = deg_ref[0, :] + deg_ref[1, :] + 1.0
        dinv = lax.rsqrt(d)
        h = jnp.maximum(
            jnp.dot(x_ref[...], w_ref[...],
                    preferred_element_type=jnp.float32) + b_ref[...], 0.0)
        db = jnp.broadcast_to(dinv[:, None], h.shape)
        h_ref[...] = h
        hs_ref[...] = h * db
        db_ref[...] = db

    o = jax.ShapeDtypeStruct((npad, 128), jnp.float32)
    return pl.pallas_call(
        body,
        grid=(nblk,),
        in_specs=[
            pl.BlockSpec((_BT, 128), lambda i: (i, 0)),
            pl.BlockSpec((128, 128), lambda i: (0, 0)),
            pl.BlockSpec((1, 128), lambda i: (0, 0)),
            pl.BlockSpec((2, _BT), lambda i: (0, i)),
        ],
        out_specs=[pl.BlockSpec((_BT, 128), lambda i: (i, 0))] * 3,
        out_shape=[o, o, o],
    )(xp, w, b, degp)


def _layer_call(p, h, x0, db, w, beta, npad):
    nblk = npad // _BT
    a1 = 1.0 - ALPHA
    b1 = 1.0 - beta

    def body(p_ref, h_ref, x0_ref, db_ref, w_ref, hn_ref, hs_ref):
        dbv = db_ref[...]
        s = p_ref[0] + p_ref[1]
        agg = dbv * s + dbv * dbv * h_ref[...]
        out = a1 * agg + ALPHA * x0_ref[...]
        m = jnp.dot(out, w_ref[...], preferred_element_type=jnp.float32)
        hn = jnp.maximum(b1 * out + beta * m, 0.0)
        hn_ref[...] = hn
        hs_ref[...] = hn * dbv

    o = jax.ShapeDtypeStruct((npad, 128), jnp.float32)
    return pl.pallas_call(
        body,
        grid=(nblk,),
        in_specs=[
            pl.BlockSpec((2, _BT, 128), lambda i: (0, i, 0)),
            pl.BlockSpec((_BT, 128), lambda i: (i, 0)),
            pl.BlockSpec((_BT, 128), lambda i: (i, 0)),
            pl.BlockSpec((_BT, 128), lambda i: (i, 0)),
            pl.BlockSpec((128, 128), lambda i: (0, 0)),
        ],
        out_specs=[pl.BlockSpec((_BT, 128), lambda i: (i, 0))] * 2,
        out_shape=[o, o],
    )(p, h, x0, db, w)


def _final_call(pp, cntp, w, b):
    def body(pp_ref, c_ref, w_ref, b_ref, o_ref):
        cnt = c_ref[0] + c_ref[1]
        s = pp_ref[0] + pp_ref[1]
        pooled = s / jnp.maximum(cnt, 1.0)[:, None]
        res = jnp.dot(pooled, w_ref[...],
                      preferred_element_type=jnp.float32) + b_ref[...]
        o_ref[...] = res[:NUM_GRAPHS]

    return pl.pallas_call(
        body,
        out_shape=jax.ShapeDtypeStruct((NUM_GRAPHS, 128), jnp.float32),
    )(pp, cntp, w, b)


def _ceil_to(v, m):
    return -(-v // m) * m


def kernel(x, edge_index, edge_attr, batch, lin0_w, lin0_b, conv_w,
           lin1_w, lin1_b):
    n = x.shape[0]
    e = edge_index.shape[1]
    npad = _ceil_to(n + 1, 2048)
    gpad = 128

    ke = _ceil_to(_ceil_to(e, _NW * _CH) // (_NW * _CH), 8)
    ep = ke * _NW * _CH
    kb = _ceil_to(_ceil_to(n, _NW * _CH) // (_NW * _CH), 8)
    nb = kb * _NW * _CH

    row = edge_index[0]
    col = edge_index[1]
    xtra = jnp.zeros((_NW, 8, 2, _CH), jnp.int32)
    rowr = jnp.concatenate(
        [row, jnp.zeros((ep - e,), jnp.int32)]).reshape(_NW, ke, _CH)
    colr = jnp.concatenate(
        [col, jnp.full((ep - e,), n, jnp.int32)]).reshape(_NW, ke, _CH)
    eidx = jnp.concatenate(
        [jnp.stack([rowr, colr], axis=2), xtra], axis=1)
    batr = jnp.concatenate(
        [batch, jnp.full((nb - n,), NUM_GRAPHS, jnp.int32)]
    ).reshape(_NW, kb, _CH)
    poolr = jnp.concatenate(
        [jnp.arange(n, dtype=jnp.int32), jnp.zeros((nb - n,), jnp.int32)]
    ).reshape(_NW, kb, _CH)
    pidx = jnp.concatenate(
        [jnp.stack([poolr, batr], axis=2), xtra], axis=1)

    zeros_n1 = jnp.zeros((npad,), jnp.float32)
    zeros_n2 = jnp.zeros((npad, 128), jnp.float32)
    zeros_g2 = jnp.zeros((gpad, 128), jnp.float32)
    xp = jnp.zeros((npad, 128), jnp.float32).at[:n].set(x)

    degp, cntp = _deg_counts_kernel(npad, ke, kb, gpad)(colr, batr, zeros_n1)
    h0, hs, db = _lin0_call(xp, lin0_w, lin0_b.reshape(1, 128), degp, npad)

    msh = _ceil_to(n + 1, 128)
    edge_scatter = _scatter_rows_kernel(msh, npad, ke)
    h = h0
    for layer in range(NUM_LAYERS):
        beta = math.log(THETA / (layer + 1) + 1.0)
        p = edge_scatter(hs, eidx, zeros_n2)
        h, hs = _layer_call(p, h, h0, db, conv_w[layer], beta, npad)

    pp = _scatter_rows_kernel(gpad, gpad, kb)(h, pidx, zeros_g2)
    return _final_call(pp, cntp, lin1_w, lin1_b.reshape(1, 128))


# speedup vs baseline: 1.3137x; 1.3137x over previous
import functools
import math

import jax
import jax.numpy as jnp
from jax import lax
from jax.experimental import pallas as pl
from jax.experimental.pallas import tpu as pltpu
from jax.experimental.pallas import tpu_sc as plsc

NUM_LAYERS = 4
ALPHA = 0.1
THETA = 0.5
NUM_GRAPHS = 64

_NC = 2
_NS = 16
_NW = _NC * _NS
_CH = 128
_HD = 64
_BT = 256


def _sc_mesh():
    return plsc.VectorSubcoreMesh(core_axis_name="c", subcore_axis_name="s")


def _deg_counts_kernel(npad, kd, kb, gpad):
    @functools.partial(
        pl.kernel,
        out_type=(jax.ShapeDtypeStruct((_NC, npad), jnp.float32),
                  jax.ShapeDtypeStruct((_NC, gpad), jnp.float32)),
        mesh=_sc_mesh(),
        scratch_types=[
            pltpu.VMEM((kd, _CH), jnp.int32),
            pltpu.VMEM((kb, _CH), jnp.int32),
            pltpu.VMEM((_CH,), jnp.float32),
            pltpu.VMEM_SHARED((npad,), jnp.float32),
            pltpu.VMEM_SHARED((gpad,), jnp.float32),
        ],
    )
    def k(col_hbm, bat_hbm, zeros_hbm, deg_out, cnt_out,
          cidx, bidx, ones_v, deg_sh, cnt_sh):
        cid = lax.axis_index("c")
        sid = lax.axis_index("s")
        wid = cid * _NS + sid
        rp = npad // _NS
        pltpu.sync_copy(zeros_hbm.at[pl.ds(sid * rp, rp)],
                        deg_sh.at[pl.ds(sid * rp, rp)])

        @pl.when(sid == 0)
        def _():
            pltpu.sync_copy(zeros_hbm.at[pl.ds(0, gpad)], cnt_sh)

        pltpu.sync_copy(col_hbm.at[wid], cidx)
        pltpu.sync_copy(bat_hbm.at[wid], bidx)
        for j in range(_CH // 16):
            ones_v[pl.ds(j * 16, 16)] = jnp.ones((16,), jnp.float32)
        plsc.subcore_barrier()

        def dbody(j, c):
            pltpu.sync_copy(ones_v, deg_sh.at[cidx.at[j]], add=True)
            return c
        lax.fori_loop(0, kd, dbody, 0)

        def bbody(j, c):
            pltpu.sync_copy(ones_v, cnt_sh.at[bidx.at[j]], add=True)
            return c
        lax.fori_loop(0, kb, bbody, 0)

        plsc.subcore_barrier()
        pltpu.sync_copy(deg_sh.at[pl.ds(sid * rp, rp)],
                        deg_out.at[cid, pl.ds(sid * rp, rp)])

        @pl.when(sid == 0)
        def _():
            pltpu.sync_copy(cnt_sh, cnt_out.at[cid])

    return k


def _scatter_rows_kernel(ash, mpad, k):
    assert k % 8 == 0 and ash % _NS == 0 and ash <= mpad

    @functools.partial(
        pl.kernel,
        out_type=jax.ShapeDtypeStruct((_NC, mpad, _HD), jnp.float32),
        mesh=_sc_mesh(),
        compiler_params=pltpu.CompilerParams(use_tc_tiling_on_sc=False),
        scratch_types=[
            pltpu.VMEM((2, 8, 2, _CH), jnp.int32),
            pltpu.VMEM((4, _CH, _HD), jnp.float32),
            pltpu.VMEM_SHARED((ash, _HD), jnp.float32),
            pltpu.SemaphoreType.DMA,
            pltpu.SemaphoreType.DMA,
            pltpu.SemaphoreType.DMA,
            pltpu.SemaphoreType.DMA,
            pltpu.SemaphoreType.DMA,
        ],
    )
    def kfn(tab_hbm, idx_hbm, zeros_hbm, out_hbm,
            idxr, rows4, agg_sh, semr0, semr1, semr2, semr3, semi):
        cid = lax.axis_index("c")
        sid = lax.axis_index("s")
        ra = ash // _NS
        pltpu.sync_copy(zeros_hbm.at[pl.ds(sid * ra, ra)],
                        agg_sh.at[pl.ds(sid * ra, ra)])
        plsc.subcore_barrier()

        semr = (semr0, semr1, semr2, semr3)

        def chunk(slot, j, gslot, gj):
            rs = j % 4
            pltpu.make_async_copy(tab_hbm.at[idxr.at[slot, j, 0]],
                                  rows4.at[rs], semr[rs]).wait()
            pltpu.sync_copy(rows4.at[rs], agg_sh.at[idxr.at[slot, j, 1]],
                            add=True)
            pltpu.async_copy(tab_hbm.at[idxr.at[gslot, gj, 0]],
                             rows4.at[rs], semr[rs])

        pltpu.sync_copy(idx_hbm.at[cid, sid, pl.ds(0, 8)], idxr.at[0])
        for j in range(4):
            pltpu.async_copy(tab_hbm.at[idxr.at[0, j, 0]],
                             rows4.at[j], semr[j])

        def body(bb, c):
            s = lax.rem(bb, 2)
            o = 1 - s
            pltpu.async_copy(idx_hbm.at[cid, sid, pl.ds((bb + 1) * 8, 8)],
                             idxr.at[o], semi)
            chunk(s, 0, s, 4)
            chunk(s, 1, s, 5)
            chunk(s, 2, s, 6)
            chunk(s, 3, s, 7)
            pltpu.make_async_copy(idx_hbm.at[cid, sid, pl.ds(0, 8)],
                                  idxr.at[o], semi).wait()
            chunk(s, 4, o, 0)
            chunk(s, 5, o, 1)
            chunk(s, 6, o, 2)
            chunk(s, 7, o, 3)
            return c
        lax.fori_loop(0, k // 8, body, 0)

        for j in range(4):
            pltpu.make_async_copy(tab_hbm.at[idxr.at[0, j, 0]],
                                  rows4.at[j], semr[j]).wait()

        plsc.subcore_barrier()
        pltpu.sync_copy(agg_sh.at[pl.ds(sid * ra, ra)],
                        out_hbm.at[cid, pl.ds(sid * ra, ra)])

    return kfn


def _lin0_call(xp, w, b, degp, npad):
    nblk = npad // _BT

    def body(x_ref, w_ref, b_ref, deg_ref, h_ref, hs_ref, db_ref):
        d = deg_ref[0, :] + deg_ref[1, :] + 1.0
        dinv = lax.rsqrt(d)
        h = jnp.maximum(
            jnp.dot(x_ref[...], w_ref[...],
                    preferred_element_type=jnp.float32) + b_ref[...], 0.0)
        db = jnp.broadcast_to(dinv[:, None], h.shape)
        hs = h * db
        h_ref[...] = h
        hs_ref[0] = hs[:, :_HD]
        hs_ref[1] = hs[:, _HD:]
        db_ref[...] = db

    return pl.pallas_call(
        body,
        grid=(nblk,),
        in_specs=[
            pl.BlockSpec((_BT, 128), lambda i: (i, 0)),
            pl.BlockSpec((128, 128), lambda i: (0, 0)),
            pl.BlockSpec((1, 128), lambda i: (0, 0)),
            pl.BlockSpec((2, _BT), lambda i: (0, i)),
        ],
        out_specs=[
            pl.BlockSpec((_BT, 128), lambda i: (i, 0)),
            pl.BlockSpec((2, _BT, _HD), lambda i: (0, i, 0)),
            pl.BlockSpec((_BT, 128), lambda i: (i, 0)),
        ],
        out_shape=[
            jax.ShapeDtypeStruct((npad, 128), jnp.float32),
            jax.ShapeDtypeStruct((2, npad, _HD), jnp.float32),
            jax.ShapeDtypeStruct((npad, 128), jnp.float32),
        ],
    )(xp, w, b, degp)


def _layer_call(p, h, x0, db, w, beta, npad, last):
    nblk = npad // _BT
    a1 = 1.0 - ALPHA
    b1 = 1.0 - beta

    def body(p_ref, h_ref, x0_ref, db_ref, w_ref, *out_refs):
        dbv = db_ref[...]
        s = jnp.concatenate([p_ref[0], p_ref[1]], axis=-1)
        agg = dbv * s + dbv * dbv * h_ref[...]
        out = a1 * agg + ALPHA * x0_ref[...]
        m = jnp.dot(out, w_ref[...], preferred_element_type=jnp.float32)
        hn = jnp.maximum(b1 * out + beta * m, 0.0)
        if last:
            h2_ref, = out_refs
            h2_ref[0] = hn[:, :_HD]
            h2_ref[1] = hn[:, _HD:]
        else:
            hn_ref, hs_ref = out_refs
            hs = hn * dbv
            hn_ref[...] = hn
            hs_ref[0] = hs[:, :_HD]
            hs_ref[1] = hs[:, _HD:]

    split = jax.ShapeDtypeStruct((2, npad, _HD), jnp.float32)
    split_spec = pl.BlockSpec((2, _BT, _HD), lambda i: (0, i, 0))
    full = jax.ShapeDtypeStruct((npad, 128), jnp.float32)
    full_spec = pl.BlockSpec((_BT, 128), lambda i: (i, 0))
    return pl.pallas_call(
        body,
        grid=(nblk,),
        in_specs=[split_spec, full_spec, full_spec, full_spec,
                  pl.BlockSpec((128, 128), lambda i: (0, 0))],
        out_specs=[split_spec] if last else [full_spec, split_spec],
        out_shape=[split] if last else [full, split],
    )(p, h, x0, db, w)


def _final_call(pp, cntp, w, b):
    def body(pp_ref, c_ref, w_ref, b_ref, o_ref):
        cnt = c_ref[0] + c_ref[1]
        s = jnp.concatenate([pp_ref[0], pp_ref[1]], axis=-1)
        pooled = s / jnp.maximum(cnt, 1.0)[:, None]
        res = jnp.dot(pooled, w_ref[...],
                      preferred_element_type=jnp.float32) + b_ref[...]
        o_ref[...] = res[:NUM_GRAPHS]

    return pl.pallas_call(
        body,
        out_shape=jax.ShapeDtypeStruct((NUM_GRAPHS, 128), jnp.float32),
    )(pp, cntp, w, b)


def _ceil_to(v, m):
    return -(-v // m) * m


def kernel(x, edge_index, edge_attr, batch, lin0_w, lin0_b, conv_w,
           lin1_w, lin1_b):
    n = x.shape[0]
    e = edge_index.shape[1]
    npad = _ceil_to(n + 1, 2048)
    gpad = 128

    row = edge_index[0]
    col = edge_index[1]

    ka = _ceil_to(e, _NW * _CH) // (_NW * _CH)
    ea = ka * _NW * _CH
    kba = _ceil_to(n, _NW * _CH) // (_NW * _CH)
    nba = kba * _NW * _CH
    cola = jnp.concatenate(
        [col, jnp.full((ea - e,), n, jnp.int32)]).reshape(_NW, ka, _CH)
    bata = jnp.concatenate(
        [batch, jnp.full((nba - n,), NUM_GRAPHS, jnp.int32)]
    ).reshape(_NW, kba, _CH)

    ke = _ceil_to(_ceil_to(e, _NS * _CH) // (_NS * _CH), 8)
    ep = ke * _NS * _CH
    kb = _ceil_to(_ceil_to(n, _NS * _CH) // (_NS * _CH), 8)
    nb = kb * _NS * _CH
    xtra = jnp.zeros((_NS, 8, 2, _CH), jnp.int32)
    plane_off = jnp.stack(
        [jnp.zeros((2, 1), jnp.int32),
         jnp.array([[npad], [0]], jnp.int32)])
    rowr = jnp.concatenate(
        [row, jnp.zeros((ep - e,), jnp.int32)]).reshape(_NS, ke, _CH)
    colr = jnp.concatenate(
        [col, jnp.full((ep - e,), n, jnp.int32)]).reshape(_NS, ke, _CH)
    eidx0 = jnp.concatenate(
        [jnp.stack([rowr, colr], axis=2), xtra], axis=1)
    eidx = eidx0[None] + plane_off[:, None, None]
    poolr = jnp.concatenate(
        [jnp.arange(n, dtype=jnp.int32), jnp.zeros((nb - n,), jnp.int32)]
    ).reshape(_NS, kb, _CH)
    batr = jnp.concatenate(
        [batch, jnp.full((nb - n,), NUM_GRAPHS, jnp.int32)]
    ).reshape(_NS, kb, _CH)
    pidx0 = jnp.concatenate(
        [jnp.stack([poolr, batr], axis=2), xtra], axis=1)
    pidx = pidx0[None] + plane_off[:, None, None]

    zeros_n1 = jnp.zeros((npad,), jnp.float32)
    zeros_n2 = jnp.zeros((npad, _HD), jnp.float32)
    xp = jnp.zeros((npad, 128), jnp.float32).at[:n].set(x)

    degp, cntp = _deg_counts_kernel(npad, ka, kba, gpad)(cola, bata, zeros_n1)
    h0, hs, db = _lin0_call(xp, lin0_w, lin0_b.reshape(1, 128), degp, npad)

    msh = _ceil_to(n + 1, 128)
    edge_scatter = _scatter_rows_kernel(msh, npad, ke)
    h = h0
    for layer in range(NUM_LAYERS):
        beta = math.log(THETA / (layer + 1) + 1.0)
        p = edge_scatter(hs.reshape(2 * npad, _HD), eidx, zeros_n2)
        last = layer == NUM_LAYERS - 1
        res = _layer_call(p, h, h0, db, conv_w[layer], beta, npad, last)
        if last:
            h2s, = res
        else:
            h, hs = res

    pp = _scatter_rows_kernel(gpad, gpad, kb)(
        h2s.reshape(2 * npad, _HD), pidx, zeros_n2)
    return _final_call(pp, cntp, lin1_w, lin1_b.reshape(1, 128))


# speedup vs baseline: 1.9737x; 1.5024x over previous
import functools
import math

import jax
import jax.numpy as jnp
from jax import lax
from jax.experimental import pallas as pl
from jax.experimental.pallas import tpu as pltpu
from jax.experimental.pallas import tpu_sc as plsc

NUM_LAYERS = 4
ALPHA = 0.1
THETA = 0.5
NUM_GRAPHS = 64

_NC = 2
_NS = 16
_NW = _NC * _NS
_CH = 128
_BT = 256


def _sc_mesh():
    return plsc.VectorSubcoreMesh(core_axis_name="c", subcore_axis_name="s")


def _deg_counts_kernel(npad, kd, kb, gpad):
    @functools.partial(
        pl.kernel,
        out_type=(jax.ShapeDtypeStruct((_NC, npad), jnp.float32),
                  jax.ShapeDtypeStruct((_NC, gpad), jnp.float32)),
        mesh=_sc_mesh(),
        scratch_types=[
            pltpu.VMEM((kd, _CH), jnp.int32),
            pltpu.VMEM((kb, _CH), jnp.int32),
            pltpu.VMEM((_CH,), jnp.float32),
            pltpu.VMEM_SHARED((npad,), jnp.float32),
            pltpu.VMEM_SHARED((gpad,), jnp.float32),
        ],
    )
    def k(col_hbm, bat_hbm, zeros_hbm, deg_out, cnt_out,
          cidx, bidx, ones_v, deg_sh, cnt_sh):
        cid = lax.axis_index("c")
        sid = lax.axis_index("s")
        wid = cid * _NS + sid
        rp = npad // _NS
        pltpu.sync_copy(zeros_hbm.at[pl.ds(sid * rp, rp)],
                        deg_sh.at[pl.ds(sid * rp, rp)])

        @pl.when(sid == 0)
        def _():
            pltpu.sync_copy(zeros_hbm.at[pl.ds(0, gpad)], cnt_sh)

        pltpu.sync_copy(col_hbm.at[wid], cidx)
        pltpu.sync_copy(bat_hbm.at[wid], bidx)
        for j in range(_CH // 16):
            ones_v[pl.ds(j * 16, 16)] = jnp.ones((16,), jnp.float32)
        plsc.subcore_barrier()

        def dbody(j, c):
            pltpu.sync_copy(ones_v, deg_sh.at[cidx.at[j]], add=True)
            return c
        lax.fori_loop(0, kd, dbody, 0)

        def bbody(j, c):
            pltpu.sync_copy(ones_v, cnt_sh.at[bidx.at[j]], add=True)
            return c
        lax.fori_loop(0, kb, bbody, 0)

        plsc.subcore_barrier()
        pltpu.sync_copy(deg_sh.at[pl.ds(sid * rp, rp)],
                        deg_out.at[cid, pl.ds(sid * rp, rp)])

        @pl.when(sid == 0)
        def _():
            pltpu.sync_copy(cnt_sh, cnt_out.at[cid])

    return k


def _scatter_rows_kernel(mpad, k):
    @functools.partial(
        pl.kernel,
        out_type=jax.ShapeDtypeStruct((_NC, mpad, 128), jnp.float32),
        mesh=_sc_mesh(),
        scratch_types=[
            pltpu.VMEM((k, _CH), jnp.int32),
            pltpu.VMEM((k, _CH), jnp.int32),
            pltpu.VMEM((_CH, 128), jnp.float32),
            pltpu.VMEM_SHARED((mpad, 128), jnp.float32),
            pltpu.SemaphoreType.DMA,
        ],
    )
    def kfn(tab_hbm, ridx_hbm, cidx_hbm, zeros_hbm, out_hbm,
            ridx, cidx, rows_v, agg_sh, sem):
        cid = lax.axis_index("c")
        sid = lax.axis_index("s")
        wid = cid * _NS + sid
        rp = mpad // _NS
        pltpu.sync_copy(zeros_hbm.at[pl.ds(sid * rp, rp)],
                        agg_sh.at[pl.ds(sid * rp, rp)])
        pltpu.sync_copy(ridx_hbm.at[wid], ridx)
        pltpu.sync_copy(cidx_hbm.at[wid], cidx)
        plsc.subcore_barrier()

        def body(j, c):
            pltpu.async_copy(tab_hbm.at[ridx.at[j]], rows_v, sem).wait()
            pltpu.sync_copy(rows_v, agg_sh.at[cidx.at[j]], add=True)
            return c
        lax.fori_loop(0, k, body, 0)

        plsc.subcore_barrier()
        pltpu.sync_copy(agg_sh.at[pl.ds(sid * rp, rp)],
                        out_hbm.at[cid, pl.ds(sid * rp, rp)])

    return kfn


def _lin0_call(xp, w, b, degp, npad):
    nblk = npad // _BT

    def body(x_ref, w_ref, b_ref, deg_ref, h_ref, hs_ref, db_ref):
        d = deg_ref[0, :] + deg_ref[1, :] + 1.0
        dinv = lax.rsqrt(d)
        h = jnp.maximum(
            jnp.dot(x_ref[...], w_ref[...],
                    preferred_element_type=jnp.float32) + b_ref[...], 0.0)
        db = jnp.broadcast_to(dinv[:, None], h.shape)
        h_ref[...] = h
        hs_ref[...] = h * db
        db_ref[...] = db

    o = jax.ShapeDtypeStruct((npad, 128), jnp.float32)
    return pl.pallas_call(
        body,
        grid=(nblk,),
        in_specs=[
            pl.BlockSpec((_BT, 128), lambda i: (i, 0)),
            pl.BlockSpec((128, 128), lambda i: (0, 0)),
            pl.BlockSpec((1, 128), lambda i: (0, 0)),
            pl.BlockSpec((2, _BT), lambda i: (0, i)),
        ],
        out_specs=[pl.BlockSpec((_BT, 128), lambda i: (i, 0))] * 3,
        out_shape=[o, o, o],
    )(xp, w, b, degp)


def _layer_call(p, h, x0, db, w, beta, npad):
    nblk = npad // _BT
    a1 = 1.0 - ALPHA
    b1 = 1.0 - beta

    def body(p_ref, h_ref, x0_ref, db_ref, w_ref, hn_ref, hs_ref):
        dbv = db_ref[...]
        s = p_ref[0] + p_ref[1]
        agg = dbv * s + dbv * dbv * h_ref[...]
        out = a1 * agg + ALPHA * x0_ref[...]
        m = jnp.dot(out, w_ref[...], preferred_element_type=jnp.float32)
        hn = jnp.maximum(b1 * out + beta * m, 0.0)
        hn_ref[...] = hn
        hs_ref[...] = hn * dbv

    o = jax.ShapeDtypeStruct((npad, 128), jnp.float32)
    return pl.pallas_call(
        body,
        grid=(nblk,),
        in_specs=[
            pl.BlockSpec((2, _BT, 128), lambda i: (0, i, 0)),
            pl.BlockSpec((_BT, 128), lambda i: (i, 0)),
            pl.BlockSpec((_BT, 128), lambda i: (i, 0)),
            pl.BlockSpec((_BT, 128), lambda i: (i, 0)),
            pl.BlockSpec((128, 128), lambda i: (0, 0)),
        ],
        out_specs=[pl.BlockSpec((_BT, 128), lambda i: (i, 0))] * 2,
        out_shape=[o, o],
    )(p, h, x0, db, w)


def _final_call(pp, cntp, w, b):
    def body(pp_ref, c_ref, w_ref, b_ref, o_ref):
        cnt = c_ref[0] + c_ref[1]
        s = pp_ref[0] + pp_ref[1]
        pooled = s / jnp.maximum(cnt, 1.0)[:, None]
        res = jnp.dot(pooled, w_ref[...],
                      preferred_element_type=jnp.float32) + b_ref[...]
        o_ref[...] = res[:NUM_GRAPHS]

    return pl.pallas_call(
        body,
        out_shape=jax.ShapeDtypeStruct((NUM_GRAPHS, 128), jnp.float32),
    )(pp, cntp, w, b)


def _ceil_to(v, m):
    return -(-v // m) * m


def kernel(x, edge_index, edge_attr, batch, lin0_w, lin0_b, conv_w,
           lin1_w, lin1_b):
    n = x.shape[0]
    e = edge_index.shape[1]
    npad = _ceil_to(n + 1, 2048)
    gpad = 128

    ke = _ceil_to(e, _NW * _CH) // (_NW * _CH)
    ep = ke * _NW * _CH
    kb = _ceil_to(n, _NW * _CH) // (_NW * _CH)
    nb = kb * _NW * _CH

    row = edge_index[0]
    col = edge_index[1]
    rowr = jnp.concatenate(
        [row, jnp.zeros((ep - e,), jnp.int32)]).reshape(_NW, ke, _CH)
    colr = jnp.concatenate(
        [col, jnp.full((ep - e,), n, jnp.int32)]).reshape(_NW, ke, _CH)
    batr = jnp.concatenate(
        [batch, jnp.full((nb - n,), NUM_GRAPHS, jnp.int32)]
    ).reshape(_NW, kb, _CH)
    poolr = jnp.concatenate(
        [jnp.arange(n, dtype=jnp.int32), jnp.zeros((nb - n,), jnp.int32)]
    ).reshape(_NW, kb, _CH)

    zeros_n1 = jnp.zeros((npad,), jnp.float32)
    zeros_n2 = jnp.zeros((npad, 128), jnp.float32)
    zeros_g2 = jnp.zeros((gpad, 128), jnp.float32)
    xp = jnp.zeros((npad, 128), jnp.float32).at[:n].set(x)

    degp, cntp = _deg_counts_kernel(npad, ke, kb, gpad)(colr, batr, zeros_n1)
    h0, hs, db = _lin0_call(xp, lin0_w, lin0_b.reshape(1, 128), degp, npad)

    edge_scatter = _scatter_rows_kernel(npad, ke)
    h = h0
    for layer in range(NUM_LAYERS):
        beta = math.log(THETA / (layer + 1) + 1.0)
        p = edge_scatter(hs, rowr, colr, zeros_n2)
        h, hs = _layer_call(p, h, h0, db, conv_w[layer], beta, npad)

    pp = _scatter_rows_kernel(gpad, kb)(h, poolr, batr, zeros_g2)
    return _final_call(pp, cntp, lin1_w, lin1_b.reshape(1, 128))


# speedup vs baseline: 2.2741x; 1.1522x over previous
import functools
import math

import jax
import jax.numpy as jnp
from jax import lax
from jax.experimental import pallas as pl
from jax.experimental.pallas import tpu as pltpu
from jax.experimental.pallas import tpu_sc as plsc

NUM_LAYERS = 4
ALPHA = 0.1
THETA = 0.5
NUM_GRAPHS = 64

_NC = 2
_NS = 16
_NW = _NC * _NS
_CH = 128
_BT = 256


def _sc_mesh():
    return plsc.VectorSubcoreMesh(core_axis_name="c", subcore_axis_name="s")


def _deg_kernel(npad, kd):
    @functools.partial(
        pl.kernel,
        out_type=jax.ShapeDtypeStruct((_NC, npad), jnp.float32),
        mesh=_sc_mesh(),
        scratch_types=[
            pltpu.VMEM((kd, _CH), jnp.int32),
            pltpu.VMEM((_CH,), jnp.float32),
            pltpu.VMEM_SHARED((npad,), jnp.float32),
        ],
    )
    def k(col_hbm, zeros_hbm, deg_out, cidx, ones_v, deg_sh):
        cid = lax.axis_index("c")
        sid = lax.axis_index("s")
        wid = cid * _NS + sid
        rp = npad // _NS
        pltpu.sync_copy(zeros_hbm.at[pl.ds(sid * rp, rp)],
                        deg_sh.at[pl.ds(sid * rp, rp)])
        pltpu.sync_copy(col_hbm.at[wid], cidx)
        for j in range(_CH // 16):
            ones_v[pl.ds(j * 16, 16)] = jnp.ones((16,), jnp.float32)
        plsc.subcore_barrier()

        def dbody(j, c):
            pltpu.sync_copy(ones_v, deg_sh.at[cidx.at[j]], add=True)
            return c
        lax.fori_loop(0, kd, dbody, 0)

        plsc.subcore_barrier()
        pltpu.sync_copy(deg_sh.at[pl.ds(sid * rp, rp)],
                        deg_out.at[cid, pl.ds(sid * rp, rp)])

    return k


def _scatter_rows_kernel(mpad, k):
    @functools.partial(
        pl.kernel,
        out_type=jax.ShapeDtypeStruct((_NC, mpad, 128), jnp.float32),
        mesh=_sc_mesh(),
        scratch_types=[
            pltpu.VMEM((k, _CH), jnp.int32),
            pltpu.VMEM((k, _CH), jnp.int32),
            pltpu.VMEM((_CH, 128), jnp.float32),
            pltpu.VMEM_SHARED((mpad, 128), jnp.float32),
            pltpu.SemaphoreType.DMA,
        ],
    )
    def kfn(tab_hbm, ridx_hbm, cidx_hbm, zeros_hbm, out_hbm,
            ridx, cidx, rows_v, agg_sh, sem):
        cid = lax.axis_index("c")
        sid = lax.axis_index("s")
        wid = cid * _NS + sid
        rp = mpad // _NS
        pltpu.sync_copy(zeros_hbm.at[pl.ds(sid * rp, rp)],
                        agg_sh.at[pl.ds(sid * rp, rp)])
        pltpu.sync_copy(ridx_hbm.at[wid], ridx)
        pltpu.sync_copy(cidx_hbm.at[wid], cidx)
        plsc.subcore_barrier()

        def body(j, c):
            pltpu.async_copy(tab_hbm.at[ridx.at[j]], rows_v, sem).wait()
            pltpu.sync_copy(rows_v, agg_sh.at[cidx.at[j]], add=True)
            return c
        lax.fori_loop(0, k, body, 0)

        plsc.subcore_barrier()
        pltpu.sync_copy(agg_sh.at[pl.ds(sid * rp, rp)],
                        out_hbm.at[cid, pl.ds(sid * rp, rp)])

    return kfn


def _lin0_call(xp, w, b, degp, npad):
    nblk = npad // _BT

    def body(x_ref, w_ref, b_ref, deg_ref, h_ref, hs_ref, db_ref):
        d = deg_ref[0, :] + deg_ref[1, :] + 1.0
        dinv = lax.rsqrt(d)
        h = jnp.maximum(
            jnp.dot(x_ref[...], w_ref[...],
                    preferred_element_type=jnp.float32) + b_ref[...], 0.0)
        db = jnp.broadcast_to(dinv[:, None], h.shape)
        h_ref[...] = h
        hs_ref[...] = h * db
        db_ref[...] = db

    o = jax.ShapeDtypeStruct((npad, 128), jnp.float32)
    return pl.pallas_call(
        body,
        grid=(nblk,),
        in_specs=[
            pl.BlockSpec((_BT, 128), lambda i: (i, 0)),
            pl.BlockSpec((128, 128), lambda i: (0, 0)),
            pl.BlockSpec((1, 128), lambda i: (0, 0)),
            pl.BlockSpec((2, _BT), lambda i: (0, i)),
        ],
        out_specs=[pl.BlockSpec((_BT, 128), lambda i: (i, 0))] * 3,
        out_shape=[o, o, o],
    )(xp, w, b, degp)


def _layer_call(p, h, x0, db, w, beta, npad):
    nblk = npad // _BT
    a1 = 1.0 - ALPHA
    b1 = 1.0 - beta

    def body(p_ref, h_ref, x0_ref, db_ref, w_ref, hn_ref, hs_ref):
        dbv = db_ref[...]
        s = p_ref[0] + p_ref[1]
        agg = dbv * s + dbv * dbv * h_ref[...]
        out = a1 * agg + ALPHA * x0_ref[...]
        m = jnp.dot(out, w_ref[...], preferred_element_type=jnp.float32)
        hn = jnp.maximum(b1 * out + beta * m, 0.0)
        hn_ref[...] = hn
        hs_ref[...] = hn * dbv

    o = jax.ShapeDtypeStruct((npad, 128), jnp.float32)
    return pl.pallas_call(
        body,
        grid=(nblk,),
        in_specs=[
            pl.BlockSpec((2, _BT, 128), lambda i: (0, i, 0)),
            pl.BlockSpec((_BT, 128), lambda i: (i, 0)),
            pl.BlockSpec((_BT, 128), lambda i: (i, 0)),
            pl.BlockSpec((_BT, 128), lambda i: (i, 0)),
            pl.BlockSpec((128, 128), lambda i: (0, 0)),
        ],
        out_specs=[pl.BlockSpec((_BT, 128), lambda i: (i, 0))] * 2,
        out_shape=[o, o],
    )(p, h, x0, db, w)


def _pool_final_call(h, bat2, w, b, npad):
    nblk = npad // _BT

    def body(bat_ref, h_ref, w_ref, b_ref, o_ref, psum, pcnt):
        i = pl.program_id(0)

        @pl.when(i == 0)
        def _():
            psum[...] = jnp.zeros_like(psum)
            pcnt[...] = jnp.zeros_like(pcnt)

        oh = jnp.equal(
            bat_ref[0][:, None],
            lax.broadcasted_iota(jnp.int32, (_BT, NUM_GRAPHS), 1)
        ).astype(jnp.float32)
        dn = (((0,), (0,)), ((), ()))
        hv = h_ref[...]
        psum[...] += lax.dot_general(oh, hv, dn,
                                     preferred_element_type=jnp.float32)
        pcnt[...] += lax.dot_general(oh, jnp.ones_like(hv), dn,
                                     preferred_element_type=jnp.float32)

        @pl.when(i == nblk - 1)
        def _():
            pooled = psum[...] / jnp.maximum(pcnt[...], 1.0)
            o_ref[...] = jnp.dot(pooled, w_ref[...],
                                 preferred_element_type=jnp.float32) \
                + b_ref[...]

    return pl.pallas_call(
        body,
        grid=(nblk,),
        in_specs=[
            pl.BlockSpec((1, _BT), lambda i: (0, i)),
            pl.BlockSpec((_BT, 128), lambda i: (i, 0)),
            pl.BlockSpec((128, 128), lambda i: (0, 0)),
            pl.BlockSpec((1, 128), lambda i: (0, 0)),
        ],
        out_specs=pl.BlockSpec((NUM_GRAPHS, 128), lambda i: (0, 0)),
        out_shape=jax.ShapeDtypeStruct((NUM_GRAPHS, 128), jnp.float32),
        scratch_shapes=[
            pltpu.VMEM((NUM_GRAPHS, 128), jnp.float32),
            pltpu.VMEM((NUM_GRAPHS, 128), jnp.float32),
        ],
    )(bat2, h, w, b)


def _ceil_to(v, m):
    return -(-v // m) * m


def kernel(x, edge_index, edge_attr, batch, lin0_w, lin0_b, conv_w,
           lin1_w, lin1_b):
    n = x.shape[0]
    e = edge_index.shape[1]
    npad = _ceil_to(n + 1, 2048)
    gpad = 128

    ke = _ceil_to(e, _NW * _CH) // (_NW * _CH)
    ep = ke * _NW * _CH

    row = edge_index[0]
    col = edge_index[1]
    rowr = jnp.concatenate(
        [row, jnp.zeros((ep - e,), jnp.int32)]).reshape(_NW, ke, _CH)
    colr = jnp.concatenate(
        [col, jnp.full((ep - e,), n, jnp.int32)]).reshape(_NW, ke, _CH)
    bat2 = jnp.concatenate(
        [batch, jnp.full((npad - n,), NUM_GRAPHS, jnp.int32)]).reshape(1, npad)

    zeros_n1 = jnp.zeros((npad,), jnp.float32)
    zeros_n2 = jnp.zeros((npad, 128), jnp.float32)
    xp = jnp.zeros((npad, 128), jnp.float32).at[:n].set(x)

    degp = _deg_kernel(npad, ke)(colr, zeros_n1)
    h0, hs, db = _lin0_call(xp, lin0_w, lin0_b.reshape(1, 128), degp, npad)

    edge_scatter = _scatter_rows_kernel(npad, ke)
    h = h0
    for layer in range(NUM_LAYERS):
        beta = math.log(THETA / (layer + 1) + 1.0)
        p = edge_scatter(hs, rowr, colr, zeros_n2)
        h, hs = _layer_call(p, h, h0, db, conv_w[layer], beta, npad)

    return _pool_final_call(h, bat2, lin1_w, lin1_b.reshape(1, 128), npad)


# speedup vs baseline: 2.4820x; 1.0914x over previous
import functools
import math

import jax
import jax.numpy as jnp
from jax import lax
from jax.experimental import pallas as pl
from jax.experimental.pallas import tpu as pltpu
from jax.experimental.pallas import tpu_sc as plsc

NUM_LAYERS = 4
ALPHA = 0.1
THETA = 0.5
NUM_GRAPHS = 64

_NC = 2
_NS = 16
_NW = _NC * _NS
_CH = 128
_BT = 256


def _sc_mesh():
    return plsc.VectorSubcoreMesh(core_axis_name="c", subcore_axis_name="s")


def _deg_kernel(npad, kd):
    @functools.partial(
        pl.kernel,
        out_type=jax.ShapeDtypeStruct((_NC, npad), jnp.float32),
        mesh=_sc_mesh(),
        scratch_types=[
            pltpu.VMEM((kd, _CH), jnp.int32),
            pltpu.VMEM((_CH,), jnp.float32),
            pltpu.VMEM_SHARED((npad,), jnp.float32),
        ],
    )
    def k(col_hbm, zeros_hbm, deg_out, cidx, ones_v, deg_sh):
        cid = lax.axis_index("c")
        sid = lax.axis_index("s")
        wid = cid * _NS + sid
        rp = npad // _NS
        pltpu.sync_copy(zeros_hbm.at[pl.ds(sid * rp, rp)],
                        deg_sh.at[pl.ds(sid * rp, rp)])
        pltpu.sync_copy(col_hbm.at[wid], cidx)
        for j in range(_CH // 16):
            ones_v[pl.ds(j * 16, 16)] = jnp.ones((16,), jnp.float32)
        plsc.subcore_barrier()

        def dbody(j, c):
            pltpu.sync_copy(ones_v, deg_sh.at[cidx.at[j]], add=True)
            return c
        lax.fori_loop(0, kd, dbody, 0)

        plsc.subcore_barrier()
        pltpu.sync_copy(deg_sh.at[pl.ds(sid * rp, rp)],
                        deg_out.at[cid, pl.ds(sid * rp, rp)])

    return k


def _scatter_rows_kernel(mpad, k, k_light, kx, light_core):
    @functools.partial(
        pl.kernel,
        out_type=jax.ShapeDtypeStruct((_NC, mpad, 128), jnp.float32),
        mesh=_sc_mesh(),
        scratch_types=[
            pltpu.VMEM((k, _CH), jnp.int32),
            pltpu.VMEM((k, _CH), jnp.int32),
            pltpu.VMEM((2, _CH), jnp.int32),
            pltpu.VMEM((_CH, 128), jnp.float32),
            pltpu.VMEM_SHARED((mpad, 128), jnp.float32),
            pltpu.SemaphoreType.DMA,
        ],
    )
    def kfn(tab_hbm, ridx_hbm, cidx_hbm, xidx_hbm, zeros_hbm, out_hbm,
            ridx, cidx, xbuf, rows_v, agg_sh, sem):
        cid = lax.axis_index("c")
        sid = lax.axis_index("s")
        wid = cid * _NS + sid
        rp = mpad // _NS
        pltpu.sync_copy(zeros_hbm.at[pl.ds(sid * rp, rp)],
                        agg_sh.at[pl.ds(sid * rp, rp)])
        pltpu.sync_copy(ridx_hbm.at[wid], ridx)
        pltpu.sync_copy(cidx_hbm.at[wid], cidx)
        plsc.subcore_barrier()

        kk = jnp.where(cid == light_core, k_light, k)

        def body(j, c):
            pltpu.async_copy(tab_hbm.at[ridx.at[j]], rows_v, sem).wait()
            pltpu.sync_copy(rows_v, agg_sh.at[cidx.at[j]], add=True)
            return c
        lax.fori_loop(0, kk, body, 0)

        @pl.when(cid != light_core)
        def _():
            def xbody(j, c):
                pltpu.sync_copy(xidx_hbm.at[sid, j], xbuf)
                pltpu.async_copy(tab_hbm.at[xbuf.at[0]], rows_v, sem).wait()
                pltpu.sync_copy(rows_v, agg_sh.at[xbuf.at[1]], add=True)
                return c
            lax.fori_loop(0, kx, xbody, 0)

        plsc.subcore_barrier()
        pltpu.sync_copy(agg_sh.at[pl.ds(sid * rp, rp)],
                        out_hbm.at[cid, pl.ds(sid * rp, rp)])

    return kfn


def _lin0_call(xp, w, b, degp, npad):
    nblk = npad // _BT

    def body(x_ref, w_ref, b_ref, deg_ref, h_ref, hs_ref, db_ref):
        d = deg_ref[0, :] + deg_ref[1, :] + 1.0
        dinv = lax.rsqrt(d)
        h = jnp.maximum(
            jnp.dot(x_ref[...], w_ref[...],
                    preferred_element_type=jnp.float32) + b_ref[...], 0.0)
        db = jnp.broadcast_to(dinv[:, None], h.shape)
        h_ref[...] = h
        hs_ref[...] = h * db
        db_ref[...] = db

    o = jax.ShapeDtypeStruct((npad, 128), jnp.float32)
    return pl.pallas_call(
        body,
        grid=(nblk,),
        in_specs=[
            pl.BlockSpec((_BT, 128), lambda i: (i, 0)),
            pl.BlockSpec((128, 128), lambda i: (0, 0)),
            pl.BlockSpec((1, 128), lambda i: (0, 0)),
            pl.BlockSpec((2, _BT), lambda i: (0, i)),
        ],
        out_specs=[pl.BlockSpec((_BT, 128), lambda i: (i, 0))] * 3,
        out_shape=[o, o, o],
    )(xp, w, b, degp)


def _layer_call(p, h, x0, db, w, beta, npad):
    nblk = npad // _BT
    a1 = 1.0 - ALPHA
    b1 = 1.0 - beta

    def body(p_ref, h_ref, x0_ref, db_ref, w_ref, hn_ref, hs_ref):
        dbv = db_ref[...]
        s = p_ref[0] + p_ref[1]
        agg = dbv * s + dbv * dbv * h_ref[...]
        out = a1 * agg + ALPHA * x0_ref[...]
        m = jnp.dot(out, w_ref[...], preferred_element_type=jnp.float32)
        hn = jnp.maximum(b1 * out + beta * m, 0.0)
        hn_ref[...] = hn
        hs_ref[...] = hn * dbv

    o = jax.ShapeDtypeStruct((npad, 128), jnp.float32)
    return pl.pallas_call(
        body,
        grid=(nblk,),
        in_specs=[
            pl.BlockSpec((2, _BT, 128), lambda i: (0, i, 0)),
            pl.BlockSpec((_BT, 128), lambda i: (i, 0)),
            pl.BlockSpec((_BT, 128), lambda i: (i, 0)),
            pl.BlockSpec((_BT, 128), lambda i: (i, 0)),
            pl.BlockSpec((128, 128), lambda i: (0, 0)),
        ],
        out_specs=[pl.BlockSpec((_BT, 128), lambda i: (i, 0))] * 2,
        out_shape=[o, o],
    )(p, h, x0, db, w)


def _pool_final_call(h, bat2, w, b, npad):
    nblk = npad // _BT

    def body(bat_ref, h_ref, w_ref, b_ref, o_ref, psum, pcnt):
        i = pl.program_id(0)

        @pl.when(i == 0)
        def _():
            psum[...] = jnp.zeros_like(psum)
            pcnt[...] = jnp.zeros_like(pcnt)

        oh = jnp.equal(
            bat_ref[0][:, None],
            lax.broadcasted_iota(jnp.int32, (_BT, NUM_GRAPHS), 1)
        ).astype(jnp.float32)
        dn = (((0,), (0,)), ((), ()))
        hv = h_ref[...]
        psum[...] += lax.dot_general(oh, hv, dn,
                                     preferred_element_type=jnp.float32)
        pcnt[...] += lax.dot_general(oh, jnp.ones_like(hv), dn,
                                     preferred_element_type=jnp.float32)

        @pl.when(i == nblk - 1)
        def _():
            pooled = psum[...] / jnp.maximum(pcnt[...], 1.0)
            o_ref[...] = jnp.dot(pooled, w_ref[...],
                                 preferred_element_type=jnp.float32) \
                + b_ref[...]

    return pl.pallas_call(
        body,
        grid=(nblk,),
        in_specs=[
            pl.BlockSpec((1, _BT), lambda i: (0, i)),
            pl.BlockSpec((_BT, 128), lambda i: (i, 0)),
            pl.BlockSpec((128, 128), lambda i: (0, 0)),
            pl.BlockSpec((1, 128), lambda i: (0, 0)),
        ],
        out_specs=pl.BlockSpec((NUM_GRAPHS, 128), lambda i: (0, 0)),
        out_shape=jax.ShapeDtypeStruct((NUM_GRAPHS, 128), jnp.float32),
        scratch_shapes=[
            pltpu.VMEM((NUM_GRAPHS, 128), jnp.float32),
            pltpu.VMEM((NUM_GRAPHS, 128), jnp.float32),
        ],
    )(bat2, h, w, b)


def _ceil_to(v, m):
    return -(-v // m) * m


def kernel(x, edge_index, edge_attr, batch, lin0_w, lin0_b, conv_w,
           lin1_w, lin1_b):
    n = x.shape[0]
    e = edge_index.shape[1]
    npad = _ceil_to(n + 1, 2048)
    gpad = 128

    ke = _ceil_to(e, _NW * _CH) // (_NW * _CH)
    ep = ke * _NW * _CH

    row = edge_index[0]
    col = edge_index[1]
    rowr = jnp.concatenate(
        [row, jnp.zeros((ep - e,), jnp.int32)]).reshape(_NW, ke, _CH)
    colr = jnp.concatenate(
        [col, jnp.full((ep - e,), n, jnp.int32)]).reshape(_NW, ke, _CH)
    bat2 = jnp.concatenate(
        [batch, jnp.full((npad - n,), NUM_GRAPHS, jnp.int32)]).reshape(1, npad)

    light = 0
    kt = _ceil_to(e, _NS * _CH) // (_NS * _CH)
    kl = min(ke, max(0, round(kt * 0.37)))
    kx = max(0, kt - ke - kl)
    cap = _NS * (kl + ke + kx) * _CH
    rowp = jnp.concatenate([row, jnp.zeros((cap - e,), jnp.int32)])
    colp = jnp.concatenate([col, jnp.full((cap - e,), n, jnp.int32)])
    sl, sh = _NS * kl * _CH, _NS * ke * _CH

    def parts(a):
        lp = jnp.concatenate(
            [a[:sl].reshape(_NS, kl, _CH),
             jnp.zeros((_NS, ke - kl, _CH), jnp.int32)], axis=1)
        hp = a[sl:sl + sh].reshape(_NS, ke, _CH)
        xp_ = a[sl + sh:].reshape(_NS, kx, _CH)
        pair = [lp, hp] if light == 0 else [hp, lp]
        return jnp.concatenate(pair, axis=0), xp_

    rowr2, rowx = parts(rowp)
    colr2, colx = parts(colp)
    xidx = jnp.stack([rowx, colx], axis=2)

    zeros_n1 = jnp.zeros((npad,), jnp.float32)
    zeros_n2 = jnp.zeros((npad, 128), jnp.float32)
    xp = jnp.zeros((npad, 128), jnp.float32).at[:n].set(x)

    degp = _deg_kernel(npad, ke)(colr, zeros_n1)
    h0, hs, db = _lin0_call(xp, lin0_w, lin0_b.reshape(1, 128), degp, npad)

    edge_scatter = _scatter_rows_kernel(npad, ke, kl, kx, light)
    h = h0
    for layer in range(NUM_LAYERS):
        beta = math.log(THETA / (layer + 1) + 1.0)
        p = edge_scatter(hs, rowr2, colr2, xidx, zeros_n2)
        h, hs = _layer_call(p, h, h0, db, conv_w[layer], beta, npad)

    return _pool_final_call(h, bat2, lin1_w, lin1_b.reshape(1, 128), npad)


# speedup vs baseline: 2.4978x; 1.0064x over previous
import functools
import math

import jax
import jax.numpy as jnp
from jax import lax
from jax.experimental import pallas as pl
from jax.experimental.pallas import tpu as pltpu
from jax.experimental.pallas import tpu_sc as plsc

NUM_LAYERS = 4
ALPHA = 0.1
THETA = 0.5
NUM_GRAPHS = 64

_NC = 2
_NS = 16
_NW = _NC * _NS
_CH = 128
_BT = 256


def _sc_mesh():
    return plsc.VectorSubcoreMesh(core_axis_name="c", subcore_axis_name="s")


def _deg_kernel(npad, kd):
    @functools.partial(
        pl.kernel,
        out_type=jax.ShapeDtypeStruct((_NC, npad), jnp.float32),
        mesh=_sc_mesh(),
        scratch_types=[
            pltpu.VMEM((kd, _CH), jnp.int32),
            pltpu.VMEM((_CH,), jnp.float32),
            pltpu.VMEM_SHARED((npad,), jnp.float32),
        ],
    )
    def k(col_hbm, zeros_hbm, deg_out, cidx, ones_v, deg_sh):
        cid = lax.axis_index("c")
        sid = lax.axis_index("s")
        wid = cid * _NS + sid
        rp = npad // _NS
        pltpu.sync_copy(zeros_hbm.at[pl.ds(sid * rp, rp)],
                        deg_sh.at[pl.ds(sid * rp, rp)])
        pltpu.sync_copy(col_hbm.at[wid], cidx)
        for j in range(_CH // 16):
            ones_v[pl.ds(j * 16, 16)] = jnp.ones((16,), jnp.float32)
        plsc.subcore_barrier()

        def dbody(j, c):
            pltpu.sync_copy(ones_v, deg_sh.at[cidx.at[j]], add=True)
            return c
        lax.fori_loop(0, kd, dbody, 0)

        plsc.subcore_barrier()
        pltpu.sync_copy(deg_sh.at[pl.ds(sid * rp, rp)],
                        deg_out.at[cid, pl.ds(sid * rp, rp)])

    return k


def _scatter_rows_kernel(mpad, k, k_light, kx, light_core):
    @functools.partial(
        pl.kernel,
        out_type=jax.ShapeDtypeStruct((_NC, mpad, 128), jnp.float32),
        mesh=_sc_mesh(),
        scratch_types=[
            pltpu.VMEM((k, _CH), jnp.int32),
            pltpu.VMEM((k, _CH), jnp.int32),
            pltpu.VMEM((2, _CH), jnp.int32),
            pltpu.VMEM((_CH, 128), jnp.float32),
            pltpu.VMEM_SHARED((mpad, 128), jnp.float32),
            pltpu.SemaphoreType.DMA,
        ],
    )
    def kfn(tab_hbm, ridx_hbm, cidx_hbm, xidx_hbm, zeros_hbm, out_hbm,
            ridx, cidx, xbuf, rows_v, agg_sh, sem):
        cid = lax.axis_index("c")
        sid = lax.axis_index("s")
        wid = cid * _NS + sid
        rp = mpad // _NS
        pltpu.sync_copy(zeros_hbm.at[pl.ds(sid * rp, rp)],
                        agg_sh.at[pl.ds(sid * rp, rp)])
        pltpu.sync_copy(ridx_hbm.at[wid], ridx)
        pltpu.sync_copy(cidx_hbm.at[wid], cidx)
        plsc.subcore_barrier()

        kk = jnp.where(cid == light_core, k_light, k)

        def body(j, c):
            pltpu.async_copy(tab_hbm.at[ridx.at[j]], rows_v, sem).wait()
            pltpu.sync_copy(rows_v, agg_sh.at[cidx.at[j]], add=True)
            return c
        lax.fori_loop(0, kk, body, 0)

        @pl.when(cid != light_core)
        def _():
            def xbody(j, c):
                pltpu.sync_copy(xidx_hbm.at[sid, j], xbuf)
                pltpu.async_copy(tab_hbm.at[xbuf.at[0]], rows_v, sem).wait()
                pltpu.sync_copy(rows_v, agg_sh.at[xbuf.at[1]], add=True)
                return c
            lax.fori_loop(0, kx, xbody, 0)

        plsc.subcore_barrier()
        pltpu.sync_copy(agg_sh.at[pl.ds(sid * rp, rp)],
                        out_hbm.at[cid, pl.ds(sid * rp, rp)])

    return kfn


def _lin0_call(xp, w, b, degp, npad):
    nblk = npad // _BT

    def body(x_ref, w_ref, b_ref, deg_ref, h_ref, hs_ref, db_ref):
        d = deg_ref[0, :] + deg_ref[1, :] + 1.0
        dinv = lax.rsqrt(d)
        h = jnp.maximum(
            jnp.dot(x_ref[...], w_ref[...],
                    preferred_element_type=jnp.float32) + b_ref[...], 0.0)
        db = jnp.broadcast_to(dinv[:, None], h.shape)
        h_ref[...] = h
        hs_ref[...] = h * db
        db_ref[...] = db

    o = jax.ShapeDtypeStruct((npad, 128), jnp.float32)
    return pl.pallas_call(
        body,
        grid=(nblk,),
        in_specs=[
            pl.BlockSpec((_BT, 128), lambda i: (i, 0)),
            pl.BlockSpec((128, 128), lambda i: (0, 0)),
            pl.BlockSpec((1, 128), lambda i: (0, 0)),
            pl.BlockSpec((2, _BT), lambda i: (0, i)),
        ],
        out_specs=[pl.BlockSpec((_BT, 128), lambda i: (i, 0))] * 3,
        out_shape=[o, o, o],
    )(xp, w, b, degp)


def _layer_call(p, h, x0, db, w, beta, npad):
    nblk = npad // _BT
    a1 = 1.0 - ALPHA
    b1 = 1.0 - beta

    def body(p_ref, h_ref, x0_ref, db_ref, w_ref, hn_ref, hs_ref):
        dbv = db_ref[...]
        s = p_ref[0] + p_ref[1]
        agg = dbv * s + dbv * dbv * h_ref[...]
        out = a1 * agg + ALPHA * x0_ref[...]
        m = jnp.dot(out, w_ref[...], preferred_element_type=jnp.float32)
        hn = jnp.maximum(b1 * out + beta * m, 0.0)
        hn_ref[...] = hn
        hs_ref[...] = hn * dbv

    o = jax.ShapeDtypeStruct((npad, 128), jnp.float32)
    return pl.pallas_call(
        body,
        grid=(nblk,),
        in_specs=[
            pl.BlockSpec((2, _BT, 128), lambda i: (0, i, 0)),
            pl.BlockSpec((_BT, 128), lambda i: (i, 0)),
            pl.BlockSpec((_BT, 128), lambda i: (i, 0)),
            pl.BlockSpec((_BT, 128), lambda i: (i, 0)),
            pl.BlockSpec((128, 128), lambda i: (0, 0)),
        ],
        out_specs=[pl.BlockSpec((_BT, 128), lambda i: (i, 0))] * 2,
        out_shape=[o, o],
    )(p, h, x0, db, w)


def _pool_final_call(h, bat2, w, b, npad):
    nblk = npad // _BT

    def body(bat_ref, h_ref, w_ref, b_ref, o_ref, psum, pcnt):
        i = pl.program_id(0)

        @pl.when(i == 0)
        def _():
            psum[...] = jnp.zeros_like(psum)
            pcnt[...] = jnp.zeros_like(pcnt)

        oh = jnp.equal(
            bat_ref[0][:, None],
            lax.broadcasted_iota(jnp.int32, (_BT, NUM_GRAPHS), 1)
        ).astype(jnp.float32)
        dn = (((0,), (0,)), ((), ()))
        hv = h_ref[...]
        psum[...] += lax.dot_general(oh, hv, dn,
                                     preferred_element_type=jnp.float32)
        pcnt[...] += lax.dot_general(oh, jnp.ones_like(hv), dn,
                                     preferred_element_type=jnp.float32)

        @pl.when(i == nblk - 1)
        def _():
            pooled = psum[...] / jnp.maximum(pcnt[...], 1.0)
            o_ref[...] = jnp.dot(pooled, w_ref[...],
                                 preferred_element_type=jnp.float32) \
                + b_ref[...]

    return pl.pallas_call(
        body,
        grid=(nblk,),
        in_specs=[
            pl.BlockSpec((1, _BT), lambda i: (0, i)),
            pl.BlockSpec((_BT, 128), lambda i: (i, 0)),
            pl.BlockSpec((128, 128), lambda i: (0, 0)),
            pl.BlockSpec((1, 128), lambda i: (0, 0)),
        ],
        out_specs=pl.BlockSpec((NUM_GRAPHS, 128), lambda i: (0, 0)),
        out_shape=jax.ShapeDtypeStruct((NUM_GRAPHS, 128), jnp.float32),
        scratch_shapes=[
            pltpu.VMEM((NUM_GRAPHS, 128), jnp.float32),
            pltpu.VMEM((NUM_GRAPHS, 128), jnp.float32),
        ],
    )(bat2, h, w, b)


def _ceil_to(v, m):
    return -(-v // m) * m


def kernel(x, edge_index, edge_attr, batch, lin0_w, lin0_b, conv_w,
           lin1_w, lin1_b):
    n = x.shape[0]
    e = edge_index.shape[1]
    npad = _ceil_to(n + 1, 2048)
    gpad = 128

    ke = _ceil_to(e, _NW * _CH) // (_NW * _CH)
    ep = ke * _NW * _CH

    row = edge_index[0]
    col = edge_index[1]
    rowr = jnp.concatenate(
        [row, jnp.zeros((ep - e,), jnp.int32)]).reshape(_NW, ke, _CH)
    colr = jnp.concatenate(
        [col, jnp.full((ep - e,), n, jnp.int32)]).reshape(_NW, ke, _CH)
    bat2 = jnp.concatenate(
        [batch, jnp.full((npad - n,), NUM_GRAPHS, jnp.int32)]).reshape(1, npad)

    light = 1
    kt = _ceil_to(e, _NS * _CH) // (_NS * _CH)
    kl = min(ke, max(0, round(kt * 0.37)))
    kx = max(0, kt - ke - kl)
    cap = _NS * (kl + ke + kx) * _CH
    rowp = jnp.concatenate([row, jnp.zeros((cap - e,), jnp.int32)])
    colp = jnp.concatenate([col, jnp.full((cap - e,), n, jnp.int32)])
    sl, sh = _NS * kl * _CH, _NS * ke * _CH

    def parts(a):
        lp = jnp.concatenate(
            [a[:sl].reshape(_NS, kl, _CH),
             jnp.zeros((_NS, ke - kl, _CH), jnp.int32)], axis=1)
        hp = a[sl:sl + sh].reshape(_NS, ke, _CH)
        xp_ = a[sl + sh:].reshape(_NS, kx, _CH)
        pair = [lp, hp] if light == 0 else [hp, lp]
        return jnp.concatenate(pair, axis=0), xp_

    rowr2, rowx = parts(rowp)
    colr2, colx = parts(colp)
    xidx = jnp.stack([rowx, colx], axis=2)

    zeros_n1 = jnp.zeros((npad,), jnp.float32)
    zeros_n2 = jnp.zeros((npad, 128), jnp.float32)
    xp = jnp.zeros((npad, 128), jnp.float32).at[:n].set(x)

    degp = _deg_kernel(npad, ke)(colr, zeros_n1)
    h0, hs, db = _lin0_call(xp, lin0_w, lin0_b.reshape(1, 128), degp, npad)

    edge_scatter = _scatter_rows_kernel(npad, ke, kl, kx, light)
    h = h0
    for layer in range(NUM_LAYERS):
        beta = math.log(THETA / (layer + 1) + 1.0)
        p = edge_scatter(hs, rowr2, colr2, xidx, zeros_n2)
        h, hs = _layer_call(p, h, h0, db, conv_w[layer], beta, npad)

    return _pool_final_call(h, bat2, lin1_w, lin1_b.reshape(1, 128), npad)
